# Initial kernel scaffold; baseline (speedup 1.0000x reference)
import jax
import jax.numpy as jnp
from jax.experimental import pallas as pl


def kernel(model_output):
    raise NotImplementedError("write your pallas kernel here")


# speedup vs baseline: 16.7015x; 16.7015x over previous
import functools

import jax
import jax.numpy as jnp
from jax.experimental import pallas as pl
from jax.experimental.pallas import tpu as pltpu

_CONF_THRES = 0.25
_IOU_THRES = 0.45
_RATIO = 0.02
_MAX_DET = 300
_MAX_WH = 7680.0
_IMG_SIZE = 640.0

_N = 20000
_ROWS = 160
_LANES = 128
_NPAD = _ROWS * _LANES


def _nms_body(inp_ref, out_ref, ox1_ref, oy1_ref, ox2_ref, oy2_ref,
              area_ref, contrib_ref):
    f32 = jnp.float32
    conf = inp_ref[4]
    cls = jnp.zeros((_ROWS, _LANES), f32)
    for c in range(1, 80):
        s = inp_ref[4 + c]
        upd = s > conf
        cls = jnp.where(upd, f32(c), cls)
        conf = jnp.where(upd, s, conf)

    xc = inp_ref[0]
    yc = inp_ref[1]
    hw = inp_ref[2] * 0.5
    hh = inp_ref[3] * 0.5
    x1 = xc - hw
    y1 = yc - hh
    x2 = xc + hw
    y2 = yc + hh
    off = cls * _MAX_WH
    ox1 = x1 + off
    oy1 = y1 + off
    ox2 = x2 + off
    oy2 = y2 + off
    ox1_ref[...] = ox1
    oy1_ref[...] = oy1
    ox2_ref[...] = ox2
    oy2_ref[...] = oy2
    area_ref[...] = (ox2 - ox1) * (oy2 - oy1)
    boxval = (x1 + y1 + x2 + y2) * (1.0 / _IMG_SIZE)
    contrib_ref[...] = conf * (boxval + cls)

    valid = conf > _CONF_THRES
    scores0 = jnp.where(valid, conf, f32(-1.0))

    i32 = jnp.int32
    flat = (jax.lax.broadcasted_iota(i32, (_ROWS, _LANES), 0) * _LANES
            + jax.lax.broadcasted_iota(i32, (_ROWS, _LANES), 1))
    lane8 = jax.lax.broadcasted_iota(i32, (8, _LANES), 1)
    row8 = jax.lax.broadcasted_iota(i32, (8, _LANES), 0)
    lmask_iota = jax.lax.broadcasted_iota(i32, (1, _LANES), 1)

    def body(i, state):
        scores, num, pconf, pcontrib = state
        best = jnp.max(scores)
        is_valid = best > 0.0
        m1 = scores >= best
        fidx = jnp.min(jnp.where(m1, flat, jnp.int32(2**30)))
        row = fidx // _LANES
        lane = fidx - row * _LANES
        lmask = lmask_iota == lane

        def gather(ref):
            return jnp.sum(jnp.where(lmask, ref[pl.ds(row, 1), :], 0.0))

        bx1 = gather(ox1_ref)
        by1 = gather(oy1_ref)
        bx2 = gather(ox2_ref)
        by2 = gather(oy2_ref)
        barea = gather(area_ref)
        bcontrib = gather(contrib_ref)

        ix1 = jnp.maximum(bx1, ox1_ref[...])
        iy1 = jnp.maximum(by1, oy1_ref[...])
        ix2 = jnp.minimum(bx2, ox2_ref[...])
        iy2 = jnp.minimum(by2, oy2_ref[...])
        inter = jnp.maximum(ix2 - ix1, 0.0) * jnp.maximum(iy2 - iy1, 0.0)
        sup = inter > _IOU_THRES * (barea + area_ref[...] - inter + 1e-7)
        mask = flat == fidx
        scores = jnp.where(sup | mask, f32(-1.0), scores)

        num = num + jnp.where(is_valid, f32(1.0), f32(0.0))
        rec = is_valid & (i < 6)
        slotmask = (row8 == 0) & (lane8 == i) & rec
        pconf = jnp.where(slotmask, best, pconf)
        pcontrib = jnp.where(slotmask, bcontrib, pcontrib)
        return scores, num, pconf, pcontrib

    init = (scores0, f32(0.0), jnp.zeros((8, _LANES), f32),
            jnp.zeros((8, _LANES), f32))
    _, num, pconf, pcontrib = jax.lax.fori_loop(0, _MAX_DET, body, init)

    k = jnp.maximum(jnp.int32(1),
                    jnp.floor(num * f32(_RATIO)).astype(jnp.int32))
    usemask = (row8 == 0) & (lane8 < k)
    wsum = jnp.sum(jnp.where(usemask, pconf, 0.0))
    wvsum = jnp.sum(jnp.where(usemask, pcontrib, 0.0))
    target = wvsum / (2.0 * wsum)
    outv = jnp.where(num > 0.0, target, f32(0.0))
    out_ref[...] = jnp.zeros((8, _LANES), f32) + outv


@jax.jit
def kernel(model_output):
    x = model_output[0]
    xt = jnp.transpose(x)
    xt = jnp.pad(xt, ((0, 0), (0, _NPAD - _N)))
    xt = xt.reshape(84, _ROWS, _LANES)
    out = pl.pallas_call(
        _nms_body,
        out_shape=jax.ShapeDtypeStruct((8, _LANES), jnp.float32),
        scratch_shapes=[pltpu.VMEM((_ROWS, _LANES), jnp.float32)
                        for _ in range(6)],
    )(xt)
    return out[0, 0]
